# Initial kernel scaffold; baseline (speedup 1.0000x reference)
import jax
import jax.numpy as jnp
from jax.experimental import pallas as pl


def kernel(x, expert_indices, W, b, lora_A, lora_B):
    raise NotImplementedError("write your pallas kernel here")


# speedup vs baseline: 1.7514x; 1.7514x over previous
import functools

import jax
import jax.numpy as jnp
import numpy as np
from jax.experimental import pallas as pl
from jax.experimental.pallas import tpu as pltpu

N_TOKENS_ = 8192
D_MODEL_ = 4096
N_EXPERTS_ = 8
TOP_K_ = 2
RANK_ = 32
SCALING_ = 16.0 / np.sqrt(RANK_)

BM = 256
BN = 512


def _moe_kernel(idx_ref, x_ref, w_ref, a_ref, bflat_ref, b_ref, out_ref,
                h_scratch):
    nj = pl.program_id(1)

    @pl.when(nj == 0)
    def _():
        h = jax.lax.dot_general(
            x_ref[...], a_ref[...],
            (((1,), (1,)), ((), ())),
            preferred_element_type=jnp.float32)
        idx = idx_ref[...]
        eid = jax.lax.broadcasted_iota(jnp.int32, (BM, N_EXPERTS_ * RANK_), 1)
        eid = eid // RANK_
        mask = (eid == idx[:, 0:1]) | (eid == idx[:, 1:2])
        h_scratch[...] = jnp.where(mask, h * SCALING_, 0.0)

    acc = jax.lax.dot_general(
        x_ref[...], w_ref[...],
        (((1,), (1,)), ((), ())),
        preferred_element_type=jnp.float32)
    acc += jnp.dot(h_scratch[...], bflat_ref[...],
                   preferred_element_type=jnp.float32)
    out_ref[...] = acc + b_ref[...]


@jax.jit
def kernel(x, expert_indices, W, b, lora_A, lora_B):
    x_shape = x.shape
    x2 = x.reshape(-1, x_shape[-1])
    n, d = x2.shape
    idx = expert_indices.reshape(-1, expert_indices.shape[-1]).astype(jnp.int32)
    a_all = lora_A.reshape(N_EXPERTS_ * RANK_, d)
    b_flat = lora_B.transpose(0, 2, 1).reshape(N_EXPERTS_ * RANK_, d)
    b2 = b.reshape(1, d)

    grid = (n // BM, d // BN)
    out = pl.pallas_call(
        _moe_kernel,
        grid=grid,
        in_specs=[
            pl.BlockSpec((BM, TOP_K_), lambda i, j: (i, 0)),
            pl.BlockSpec((BM, d), lambda i, j: (i, 0)),
            pl.BlockSpec((BN, d), lambda i, j: (j, 0)),
            pl.BlockSpec((N_EXPERTS_ * RANK_, d), lambda i, j: (0, 0)),
            pl.BlockSpec((N_EXPERTS_ * RANK_, BN), lambda i, j: (0, j)),
            pl.BlockSpec((1, BN), lambda i, j: (0, j)),
        ],
        out_specs=pl.BlockSpec((BM, BN), lambda i, j: (i, j)),
        out_shape=jax.ShapeDtypeStruct((n, d), jnp.float32),
        scratch_shapes=[pltpu.VMEM((BM, N_EXPERTS_ * RANK_), jnp.float32)],
    )(idx, x2, W, a_all, b_flat, b2)
    return out.reshape(x_shape[:-1] + (d,))


# speedup vs baseline: 2.0347x; 1.1617x over previous
import functools

import jax
import jax.numpy as jnp
import numpy as np
from jax.experimental import pallas as pl
from jax.experimental.pallas import tpu as pltpu

N_TOKENS_ = 8192
D_MODEL_ = 4096
N_EXPERTS_ = 8
TOP_K_ = 2
RANK_ = 32
SCALING_ = 16.0 / np.sqrt(RANK_)

BM = 256
BN = 512


def _moe_kernel(idx_ref, x_ref, w_ref, a_ref, bflat_ref, b_ref, out_ref,
                h_scratch):
    nj = pl.program_id(1)

    @pl.when(nj == 0)
    def _():
        h = jax.lax.dot_general(
            x_ref[...], a_ref[...],
            (((1,), (1,)), ((), ())),
            preferred_element_type=jnp.float32)
        idx = idx_ref[...]
        eid = jax.lax.broadcasted_iota(jnp.int32, (BM, N_EXPERTS_ * RANK_), 1)
        eid = eid // RANK_
        mask = (eid == idx[:, 0:1]) | (eid == idx[:, 1:2])
        h_scratch[...] = jnp.where(mask, h * SCALING_, 0.0).astype(jnp.bfloat16)

    acc = jax.lax.dot_general(
        x_ref[...], w_ref[...],
        (((1,), (1,)), ((), ())),
        preferred_element_type=jnp.float32)
    acc += jnp.dot(h_scratch[...], bflat_ref[...],
                   preferred_element_type=jnp.float32)
    out_ref[...] = acc + b_ref[...]


@jax.jit
def kernel(x, expert_indices, W, b, lora_A, lora_B):
    x_shape = x.shape
    x2 = x.reshape(-1, x_shape[-1])
    n, d = x2.shape
    idx = expert_indices.reshape(-1, expert_indices.shape[-1]).astype(jnp.int32)
    x2 = x2.astype(jnp.bfloat16)
    W = W.astype(jnp.bfloat16)
    a_all = lora_A.reshape(N_EXPERTS_ * RANK_, d).astype(jnp.bfloat16)
    b_flat = lora_B.transpose(0, 2, 1).reshape(N_EXPERTS_ * RANK_, d).astype(jnp.bfloat16)
    b2 = b.reshape(1, d)

    grid = (n // BM, d // BN)
    out = pl.pallas_call(
        _moe_kernel,
        grid=grid,
        in_specs=[
            pl.BlockSpec((BM, TOP_K_), lambda i, j: (i, 0)),
            pl.BlockSpec((BM, d), lambda i, j: (i, 0)),
            pl.BlockSpec((BN, d), lambda i, j: (j, 0)),
            pl.BlockSpec((N_EXPERTS_ * RANK_, d), lambda i, j: (0, 0)),
            pl.BlockSpec((N_EXPERTS_ * RANK_, BN), lambda i, j: (0, j)),
            pl.BlockSpec((1, BN), lambda i, j: (0, j)),
        ],
        out_specs=pl.BlockSpec((BM, BN), lambda i, j: (i, j)),
        out_shape=jax.ShapeDtypeStruct((n, d), jnp.float32),
        scratch_shapes=[pltpu.VMEM((BM, N_EXPERTS_ * RANK_), jnp.bfloat16)],
    )(idx, x2, W, a_all, b_flat, b2)
    return out.reshape(x_shape[:-1] + (d,))


# speedup vs baseline: 3.0113x; 1.4799x over previous
import functools

import jax
import jax.numpy as jnp
import numpy as np
from jax.experimental import pallas as pl
from jax.experimental.pallas import tpu as pltpu

N_TOKENS_ = 8192
D_MODEL_ = 4096
N_EXPERTS_ = 8
TOP_K_ = 2
RANK_ = 32
SCALING_ = 16.0 / np.sqrt(RANK_)

BM = 1024
BN = 512


def _moe_kernel(idx_ref, x_ref, w_ref, a_ref, bflat_ref, b_ref, out_ref,
                h_scratch):
    nj = pl.program_id(1)

    xb = x_ref[...].astype(jnp.bfloat16)

    @pl.when(nj == 0)
    def _():
        h = jax.lax.dot_general(
            xb, a_ref[...],
            (((1,), (1,)), ((), ())),
            preferred_element_type=jnp.float32)
        idx = idx_ref[...]
        eid = jax.lax.broadcasted_iota(jnp.int32, (BM, N_EXPERTS_ * RANK_), 1)
        eid = eid // RANK_
        mask = (eid == idx[:, 0:1]) | (eid == idx[:, 1:2])
        h_scratch[...] = jnp.where(mask, h * SCALING_, 0.0).astype(jnp.bfloat16)

    acc = jax.lax.dot_general(
        xb, w_ref[...],
        (((1,), (1,)), ((), ())),
        preferred_element_type=jnp.float32)
    acc += jnp.dot(h_scratch[...], bflat_ref[...],
                   preferred_element_type=jnp.float32)
    out_ref[...] = acc + b_ref[...]


@jax.jit
def kernel(x, expert_indices, W, b, lora_A, lora_B):
    x_shape = x.shape
    x2 = x.reshape(-1, x_shape[-1])
    n, d = x2.shape
    idx = expert_indices.reshape(-1, expert_indices.shape[-1]).astype(jnp.int32)
    W = W.astype(jnp.bfloat16)
    a_all = lora_A.reshape(N_EXPERTS_ * RANK_, d).astype(jnp.bfloat16)
    b_flat = lora_B.transpose(0, 2, 1).reshape(N_EXPERTS_ * RANK_, d).astype(jnp.bfloat16)
    b2 = b.reshape(1, d)

    grid = (n // BM, d // BN)
    out = pl.pallas_call(
        _moe_kernel,
        grid=grid,
        in_specs=[
            pl.BlockSpec((BM, TOP_K_), lambda i, j: (i, 0)),
            pl.BlockSpec((BM, d), lambda i, j: (i, 0)),
            pl.BlockSpec((BN, d), lambda i, j: (j, 0)),
            pl.BlockSpec((N_EXPERTS_ * RANK_, d), lambda i, j: (0, 0)),
            pl.BlockSpec((N_EXPERTS_ * RANK_, BN), lambda i, j: (0, j)),
            pl.BlockSpec((1, BN), lambda i, j: (0, j)),
        ],
        out_specs=pl.BlockSpec((BM, BN), lambda i, j: (i, j)),
        out_shape=jax.ShapeDtypeStruct((n, d), jnp.float32),
        scratch_shapes=[pltpu.VMEM((BM, N_EXPERTS_ * RANK_), jnp.bfloat16)],
    )(idx, x2, W, a_all, b_flat, b2)
    return out.reshape(x_shape[:-1] + (d,))


# speedup vs baseline: 3.3194x; 1.1023x over previous
import functools

import jax
import jax.numpy as jnp
import numpy as np
from jax.experimental import pallas as pl
from jax.experimental.pallas import tpu as pltpu

N_TOKENS_ = 8192
D_MODEL_ = 4096
N_EXPERTS_ = 8
TOP_K_ = 2
RANK_ = 32
SCALING_ = 16.0 / np.sqrt(RANK_)

BM = 1024
BN = 512


def _moe_kernel(idx_ref, x_ref, w_ref, a_ref, bflat_ref, b_ref, out_ref,
                h_scratch, xb_scratch):
    nj = pl.program_id(1)

    @pl.when(nj == 0)
    def _():
        xb = x_ref[...].astype(jnp.bfloat16)
        xb_scratch[...] = xb
        h = jax.lax.dot_general(
            xb, a_ref[...],
            (((1,), (1,)), ((), ())),
            preferred_element_type=jnp.float32)
        idx = idx_ref[...]
        eid = jax.lax.broadcasted_iota(jnp.int32, (BM, N_EXPERTS_ * RANK_), 1)
        eid = eid // RANK_
        mask = (eid == idx[:, 0:1]) | (eid == idx[:, 1:2])
        h_scratch[...] = jnp.where(mask, h * SCALING_, 0.0).astype(jnp.bfloat16)

    acc = jax.lax.dot_general(
        xb_scratch[...], w_ref[...],
        (((1,), (1,)), ((), ())),
        preferred_element_type=jnp.float32)
    acc += jnp.dot(h_scratch[...], bflat_ref[...],
                   preferred_element_type=jnp.float32)
    out_ref[...] = acc + b_ref[...]


@jax.jit
def kernel(x, expert_indices, W, b, lora_A, lora_B):
    x_shape = x.shape
    x2 = x.reshape(-1, x_shape[-1])
    n, d = x2.shape
    idx = expert_indices.reshape(-1, expert_indices.shape[-1]).astype(jnp.int32)
    W = W.astype(jnp.bfloat16)
    a_all = lora_A.reshape(N_EXPERTS_ * RANK_, d).astype(jnp.bfloat16)
    b_flat = lora_B.transpose(0, 2, 1).reshape(N_EXPERTS_ * RANK_, d).astype(jnp.bfloat16)
    b2 = b.reshape(1, d)

    grid = (n // BM, d // BN)
    out = pl.pallas_call(
        _moe_kernel,
        grid=grid,
        in_specs=[
            pl.BlockSpec((BM, TOP_K_), lambda i, j: (i, 0)),
            pl.BlockSpec((BM, d), lambda i, j: (i, 0)),
            pl.BlockSpec((BN, d), lambda i, j: (j, 0)),
            pl.BlockSpec((N_EXPERTS_ * RANK_, d), lambda i, j: (0, 0)),
            pl.BlockSpec((N_EXPERTS_ * RANK_, BN), lambda i, j: (0, j)),
            pl.BlockSpec((1, BN), lambda i, j: (0, j)),
        ],
        out_specs=pl.BlockSpec((BM, BN), lambda i, j: (i, j)),
        out_shape=jax.ShapeDtypeStruct((n, d), jnp.float32),
        scratch_shapes=[pltpu.VMEM((BM, N_EXPERTS_ * RANK_), jnp.bfloat16),
                        pltpu.VMEM((BM, d), jnp.bfloat16)],
    )(idx, x2, W, a_all, b_flat, b2)
    return out.reshape(x_shape[:-1] + (d,))
